# baseline (speedup 1.0000x reference)
import jax
import jax.numpy as jnp
from jax.experimental import pallas as pl
from jax.experimental.pallas import tpu as pltpu

_EPS = 1e-5
_INV_SQRT2 = 0.7071067811865476


def _gelu(v):
    return 0.5 * v * (1.0 + jax.lax.erf(v * _INV_SQRT2))


def _batch_block(N, cap=8):
    best = 1
    for d in range(2, min(N, cap) + 1):
        if N % d == 0 and (N // d) >= 2:
            best = d
    return best


def _make_body(Nb, H, W, Cin, Cmid, Cout):
    HW = H * W
    M = Nb * HW
    S = HW + W
    P = W + Nb * S
    B = P + 2 * W
    K3 = 3 * Cin

    def body(x_ref, w1a_ref, w1b_ref, w2a_ref, w2b_ref,
             g1a_ref, b1a_ref, g1b_ref, b1b_ref,
             g2a_ref, b2a_ref, g2b_ref, b2b_ref,
             emb_ref, out_ref, ext_ref):
        wrow = jax.lax.rem(
            jax.lax.broadcasted_iota(jnp.int32, (HW, 1), 0), W)
        mask_wlo = (wrow >= 1).astype(jnp.bfloat16)
        mask_whi = (wrow <= W - 2).astype(jnp.bfloat16)

        def store_slabs(b, yb):
            Ci = yb.shape[1]
            a0 = yb.astype(jnp.bfloat16)
            zrow = jnp.zeros((1, Ci), jnp.bfloat16)
            am = jnp.concatenate([zrow, a0[:-1, :]], axis=0) * mask_wlo
            ap = jnp.concatenate([a0[1:, :], zrow], axis=0) * mask_whi
            r0 = 2 * W + b * S
            ext_ref[pl.ds(r0, HW), pl.ds(0, Ci)] = am
            ext_ref[pl.ds(r0, HW), pl.ds(Ci, Ci)] = a0
            ext_ref[pl.ds(r0, HW), pl.ds(2 * Ci, Ci)] = ap

        def conv3x3(Ci, w_ref):
            acc = None
            for dy in range(3):
                lhs = ext_ref[pl.ds(dy * W, P), pl.ds(0, 3 * Ci)]
                d = jnp.dot(lhs, w_ref[pl.ds(dy * 3 * Ci, 3 * Ci), :],
                            preferred_element_type=jnp.float32)
                acc = d if acc is None else acc + d
            return acc

        def group_norm(h, g_ref, b_ref, post):
            gamma = g_ref[...]
            beta = b_ref[...]
            inv_n = 1.0 / float(h.shape[1] * HW)
            for b in range(Nb):
                blk = h[W + b * S:W + b * S + HW, :]
                mean = jnp.sum(blk, keepdims=True) * inv_n
                cent = blk - mean
                var = jnp.sum(cent * cent, keepdims=True) * inv_n
                y = cent * jax.lax.rsqrt(var + _EPS) * gamma + beta
                post(b, y)

        zhalo = jnp.zeros((2 * W, K3), jnp.bfloat16)
        ext_ref[pl.ds(0, 2 * W), :] = zhalo
        for b in range(Nb):
            ext_ref[pl.ds(2 * W + b * S + HW, W), :] = zhalo[:W]
        ext_ref[pl.ds(2 * W + Nb * S, B - 2 * W - Nb * S), :] = (
            zhalo[:B - 2 * W - Nb * S])

        for b in range(Nb):
            store_slabs(b, x_ref[pl.ds(b * HW, HW), :])

        h = conv3x3(Cin, w1a_ref)
        group_norm(h, g1a_ref, b1a_ref,
                   lambda b, y: store_slabs(b, _gelu(y)))
        h = conv3x3(Cin, w1b_ref)
        group_norm(
            h, g1b_ref, b1b_ref,
            lambda b, y: store_slabs(
                b, _gelu(x_ref[pl.ds(b * HW, HW), :] + y)))

        m = conv3x3(Cin, w2a_ref)
        group_norm(m, g2a_ref, b2a_ref,
                   lambda b, y: store_slabs(b, _gelu(y)))
        o = conv3x3(Cmid, w2b_ref)

        def final(b, y):
            out_ref[pl.ds(b * HW, HW), :] = (
                y + emb_ref[b:b + 1, :]).astype(out_ref.dtype)
        group_norm(o, g2b_ref, b2b_ref, final)

    return body, B, K3


def _upsample2x(x):
    N, C, Hin, Win = x.shape
    Hout, Wout = 2 * Hin, 2 * Win

    def coords(n_in, n_out):
        src = jnp.arange(n_out, dtype=jnp.float32) * (n_in - 1) / (n_out - 1)
        lo = jnp.clip(jnp.floor(src).astype(jnp.int32), 0, n_in - 2)
        frac = src - lo.astype(jnp.float32)
        return lo, lo + 1, frac

    hlo, hhi, fh = coords(Hin, Hout)
    wlo, whi, fw = coords(Win, Wout)
    top = (x[:, :, hlo, :] * (1.0 - fh)[None, None, :, None]
           + x[:, :, hhi, :] * fh[None, None, :, None])
    return (top[:, :, :, wlo] * (1.0 - fw)[None, None, None, :]
            + top[:, :, :, whi] * fw[None, None, None, :])


@jax.jit
def kernel(w1a, w1b, w2a, w2b, g1a, b1a, g1b, b1b,
           g2a, b2a, g2b, b2b, wlin, blin, x, skip_x, embeddings):
    xu = _upsample2x(x)
    xc = jnp.concatenate([skip_x, xu], axis=1)
    N, Cin, H, W = xc.shape
    HW = H * W

    xf = jnp.transpose(xc, (0, 2, 3, 1)).reshape(N * HW, Cin)
    xf = xf.astype(jnp.float32)

    Cmid = w2a.shape[-1]
    Cout = w2b.shape[-1]

    wb1a = w1a.reshape(9 * Cin, Cin).astype(jnp.bfloat16)
    wb1b = w1b.reshape(9 * Cin, Cin).astype(jnp.bfloat16)
    wb2a = w2a.reshape(9 * Cin, Cmid).astype(jnp.bfloat16)
    wb2b = w2b.reshape(9 * Cmid, Cout).astype(jnp.bfloat16)

    ga1a = g1a.reshape(1, Cin)
    bb1a = b1a.reshape(1, Cin)
    ga1b = g1b.reshape(1, Cin)
    bb1b = b1b.reshape(1, Cin)
    ga2a = g2a.reshape(1, Cmid)
    bb2a = b2a.reshape(1, Cmid)
    ga2b = g2b.reshape(1, Cout)
    bb2b = b2b.reshape(1, Cout)

    e = embeddings.astype(jnp.float32)
    e = e * jax.nn.sigmoid(e)
    emb = e @ wlin + blin

    Nb = _batch_block(N)
    M = Nb * HW
    body, B, K3 = _make_body(Nb, H, W, Cin, Cmid, Cout)

    def full(a):
        nd = a.ndim
        return pl.BlockSpec(a.shape, lambda b: (0,) * nd)

    out = pl.pallas_call(
        body,
        out_shape=jax.ShapeDtypeStruct((N * HW, Cout), jnp.float32),
        grid=(N // Nb,),
        in_specs=[
            pl.BlockSpec((M, Cin), lambda b: (b, 0)),
            full(wb1a), full(wb1b), full(wb2a), full(wb2b),
            full(ga1a), full(bb1a), full(ga1b), full(bb1b),
            full(ga2a), full(bb2a), full(ga2b), full(bb2b),
            pl.BlockSpec((Nb, Cout), lambda b: (b, 0)),
        ],
        out_specs=pl.BlockSpec((M, Cout), lambda b: (b, 0)),
        scratch_shapes=[pltpu.VMEM((B, K3), jnp.bfloat16)],
        compiler_params=pltpu.CompilerParams(
            dimension_semantics=("parallel",)),
    )(xf, wb1a, wb1b, wb2a, wb2b,
      ga1a, bb1a, ga1b, bb1b, ga2a, bb2a, ga2b, bb2b, emb)

    return jnp.transpose(out.reshape(N, HW, Cout), (0, 2, 1)).reshape(
        N, Cout, H, W)


# speedup vs baseline: 1.2221x; 1.2221x over previous
import jax
import jax.numpy as jnp
from jax.experimental import pallas as pl
from jax.experimental.pallas import tpu as pltpu

_EPS = 1e-5
_INV_SQRT2 = 0.7071067811865476


def _gelu(v):
    return 0.5 * v * (1.0 + jax.lax.erf(v * _INV_SQRT2))


def _batch_block(N, cap=8):
    best = 1
    for d in range(2, min(N, cap) + 1):
        if N % d == 0 and (N // d) >= 2:
            best = d
    return best


def _make_body(Nb, H, W, Cx, Cs, Cmid, Cout):
    Cin = Cx + Cs
    HW = H * W
    HWin = (H // 2) * (W // 2)
    M = Nb * HW
    S = HW + W
    P = W + Nb * S
    B = P + 2 * W
    K3 = 3 * Cin

    def body(x_ref, skip_ref, k2_ref, w1a_ref, w1b_ref, w2a_ref, w2b_ref,
             g1a_ref, b1a_ref, g1b_ref, b1b_ref,
             g2a_ref, b2a_ref, g2b_ref, b2b_ref,
             emb_ref, out_ref, ext_ref, xs_ref):
        wrow = jax.lax.rem(
            jax.lax.broadcasted_iota(jnp.int32, (HW, 1), 0), W)
        mask_wlo = (wrow >= 1).astype(jnp.bfloat16)
        mask_whi = (wrow <= W - 2).astype(jnp.bfloat16)

        def store_slabs(b, yb):
            Ci = yb.shape[1]
            a0 = yb.astype(jnp.bfloat16)
            zrow = jnp.zeros((1, Ci), jnp.bfloat16)
            am = jnp.concatenate([zrow, a0[:-1, :]], axis=0) * mask_wlo
            ap = jnp.concatenate([a0[1:, :], zrow], axis=0) * mask_whi
            r0 = 2 * W + b * S
            ext_ref[pl.ds(r0, HW), pl.ds(0, Ci)] = am
            ext_ref[pl.ds(r0, HW), pl.ds(Ci, Ci)] = a0
            ext_ref[pl.ds(r0, HW), pl.ds(2 * Ci, Ci)] = ap

        def conv3x3(Ci, w_ref):
            acc = None
            for dy in range(3):
                lhs = ext_ref[pl.ds(dy * W, P), pl.ds(0, 3 * Ci)]
                d = jnp.dot(lhs, w_ref[pl.ds(dy * 3 * Ci, 3 * Ci), :],
                            preferred_element_type=jnp.float32)
                acc = d if acc is None else acc + d
            return acc

        def group_norm(h, g_ref, b_ref, post):
            gamma = g_ref[...]
            beta = b_ref[...]
            inv_n = 1.0 / float(h.shape[1] * HW)
            for b in range(Nb):
                blk = h[W + b * S:W + b * S + HW, :]
                mean = jnp.sum(blk, keepdims=True) * inv_n
                cent = blk - mean
                var = jnp.sum(cent * cent, keepdims=True) * inv_n
                y = cent * jax.lax.rsqrt(var + _EPS) * gamma + beta
                post(b, y)

        zhalo = jnp.zeros((2 * W, K3), jnp.bfloat16)
        ext_ref[pl.ds(0, 2 * W), :] = zhalo
        for b in range(Nb):
            ext_ref[pl.ds(2 * W + b * S + HW, W), :] = zhalo[:W]
        ext_ref[pl.ds(2 * W + Nb * S, W), :] = zhalo[:W]

        k2 = k2_ref[...]
        for b in range(Nb):
            sk = jnp.transpose(skip_ref[b])
            xt = jnp.transpose(x_ref[pl.ds(b * Cx, Cx), :])
            up = jnp.dot(k2, xt, preferred_element_type=jnp.float32)
            act = jnp.concatenate([sk, up], axis=1)
            xs_ref[pl.ds(b * HW, HW), :] = act
            store_slabs(b, act)

        h = conv3x3(Cin, w1a_ref)
        group_norm(h, g1a_ref, b1a_ref,
                   lambda b, y: store_slabs(b, _gelu(y)))
        h = conv3x3(Cin, w1b_ref)
        group_norm(
            h, g1b_ref, b1b_ref,
            lambda b, y: store_slabs(
                b, _gelu(xs_ref[pl.ds(b * HW, HW), :] + y)))

        m = conv3x3(Cin, w2a_ref)
        group_norm(m, g2a_ref, b2a_ref,
                   lambda b, y: store_slabs(b, _gelu(y)))
        o = conv3x3(Cmid, w2b_ref)

        def final(b, y):
            out_ref[pl.ds(b * HW, HW), :] = (
                y + emb_ref[b:b + 1, :]).astype(out_ref.dtype)
        group_norm(o, g2b_ref, b2b_ref, final)

    return body, B, K3


def _bilinear_matrix(Hin, Win, H, W):
    def axis_w(n_in, n_out):
        src = jnp.arange(n_out, dtype=jnp.float32) * (n_in - 1) / (n_out - 1)
        lo = jnp.clip(jnp.floor(src).astype(jnp.int32), 0, n_in - 2)
        frac = src - lo.astype(jnp.float32)
        i = jnp.arange(n_in)[None, :]
        wlo = (i == lo[:, None]) * (1.0 - frac)[:, None]
        whi = (i == (lo + 1)[:, None]) * frac[:, None]
        return wlo + whi

    uh = axis_w(Hin, H)
    uw = axis_w(Win, W)
    k2 = uh[:, None, :, None] * uw[None, :, None, :]
    return k2.reshape(H * W, Hin * Win)


@jax.jit
def kernel(w1a, w1b, w2a, w2b, g1a, b1a, g1b, b1b,
           g2a, b2a, g2b, b2b, wlin, blin, x, skip_x, embeddings):
    N, Cx, Hin, Win = x.shape
    _, Cs, H, W = skip_x.shape
    Cin = Cx + Cs
    HW = H * W
    HWin = Hin * Win

    x2d = x.reshape(N * Cx, HWin).astype(jnp.float32)
    skip3 = skip_x.reshape(N, Cs, HW).astype(jnp.float32)
    k2 = _bilinear_matrix(Hin, Win, H, W)

    Cmid = w2a.shape[-1]
    Cout = w2b.shape[-1]

    wb1a = w1a.reshape(9 * Cin, Cin).astype(jnp.bfloat16)
    wb1b = w1b.reshape(9 * Cin, Cin).astype(jnp.bfloat16)
    wb2a = w2a.reshape(9 * Cin, Cmid).astype(jnp.bfloat16)
    wb2b = w2b.reshape(9 * Cmid, Cout).astype(jnp.bfloat16)

    ga1a = g1a.reshape(1, Cin)
    bb1a = b1a.reshape(1, Cin)
    ga1b = g1b.reshape(1, Cin)
    bb1b = b1b.reshape(1, Cin)
    ga2a = g2a.reshape(1, Cmid)
    bb2a = b2a.reshape(1, Cmid)
    ga2b = g2b.reshape(1, Cout)
    bb2b = b2b.reshape(1, Cout)

    e = embeddings.astype(jnp.float32)
    e = e * jax.nn.sigmoid(e)
    emb = e @ wlin + blin

    Nb = _batch_block(N)
    M = Nb * HW
    body, B, K3 = _make_body(Nb, H, W, Cx, Cs, Cmid, Cout)

    def full(a):
        nd = a.ndim
        return pl.BlockSpec(a.shape, lambda b: (0,) * nd)

    out = pl.pallas_call(
        body,
        out_shape=jax.ShapeDtypeStruct((N * HW, Cout), jnp.float32),
        grid=(N // Nb,),
        in_specs=[
            pl.BlockSpec((Nb * Cx, HWin), lambda b: (b, 0)),
            pl.BlockSpec((Nb, Cs, HW), lambda b: (b, 0, 0)),
            full(k2),
            full(wb1a), full(wb1b), full(wb2a), full(wb2b),
            full(ga1a), full(bb1a), full(ga1b), full(bb1b),
            full(ga2a), full(bb2a), full(ga2b), full(bb2b),
            pl.BlockSpec((Nb, Cout), lambda b: (b, 0)),
        ],
        out_specs=pl.BlockSpec((M, Cout), lambda b: (b, 0)),
        scratch_shapes=[pltpu.VMEM((B, K3), jnp.bfloat16),
                        pltpu.VMEM((M, Cin), jnp.float32)],
        compiler_params=pltpu.CompilerParams(
            dimension_semantics=("parallel",)),
    )(x2d, skip3, k2, wb1a, wb1b, wb2a, wb2b,
      ga1a, bb1a, ga1b, bb1b, ga2a, bb2a, ga2b, bb2b, emb)

    return jnp.transpose(out.reshape(N, HW, Cout), (0, 2, 1)).reshape(
        N, Cout, H, W)


# speedup vs baseline: 1.3456x; 1.1010x over previous
import jax
import jax.numpy as jnp
from jax.experimental import pallas as pl
from jax.experimental.pallas import tpu as pltpu

_EPS = 1e-5
_INV_SQRT2 = 0.7071067811865476


def _gelu(v):
    return 0.5 * v * (1.0 + jax.lax.erf(v * _INV_SQRT2))


def _batch_block(N, cap=8):
    best = 1
    for d in range(2, min(N, cap) + 1):
        if N % d == 0 and (N // d) >= 2:
            best = d
    return best


def _make_body(Nb, H, W, Cx, Cs, Cmid, Cout):
    Cin = Cx + Cs
    HW = H * W
    HWin = (H // 2) * (W // 2)
    M = Nb * HW
    S = HW + W
    P = W + Nb * S
    B = P + 2 * W
    K3 = 3 * Cin

    def body(x_ref, skip_ref, k2_ref, w1a_ref, w1b_ref, w2a_ref, w2b_ref,
             g1a_ref, b1a_ref, g1b_ref, b1b_ref,
             g2a_ref, b2a_ref, g2b_ref, b2b_ref,
             emb_ref, out_ref, ext_ref, xs_ref):
        wrow = jax.lax.rem(
            jax.lax.broadcasted_iota(jnp.int32, (HW, 1), 0), W)
        mask_wlo = (wrow >= 1).astype(jnp.bfloat16)
        mask_whi = (wrow <= W - 2).astype(jnp.bfloat16)

        def store_slabs(b, yb):
            Ci = yb.shape[1]
            a0 = yb.astype(jnp.bfloat16)
            zrow = jnp.zeros((1, Ci), jnp.bfloat16)
            am = jnp.concatenate([zrow, a0[:-1, :]], axis=0) * mask_wlo
            ap = jnp.concatenate([a0[1:, :], zrow], axis=0) * mask_whi
            r0 = 2 * W + b * S
            ext_ref[pl.ds(r0, HW), pl.ds(0, Ci)] = am
            ext_ref[pl.ds(r0, HW), pl.ds(Ci, Ci)] = a0
            ext_ref[pl.ds(r0, HW), pl.ds(2 * Ci, Ci)] = ap

        def conv3x3(Ci, w_ref):
            acc = None
            for dy in range(3):
                lhs = ext_ref[pl.ds(dy * W, P), pl.ds(0, 3 * Ci)]
                d = jnp.dot(lhs, w_ref[pl.ds(dy * 3 * Ci, 3 * Ci), :],
                            preferred_element_type=jnp.float32)
                acc = d if acc is None else acc + d
            return acc

        def group_norm(h, g_ref, b_ref, post):
            gamma = g_ref[...]
            beta = b_ref[...]
            inv_n = 1.0 / float(h.shape[1] * HW)
            for b in range(Nb):
                blk = h[W + b * S:W + b * S + HW, :]
                mean = jnp.sum(blk, keepdims=True) * inv_n
                var = jnp.sum(blk * blk, keepdims=True) * inv_n - mean * mean
                a = jax.lax.rsqrt(var + _EPS) * gamma
                y = blk * a + (beta - mean * a)
                post(b, y)

        zhalo = jnp.zeros((2 * W, K3), jnp.bfloat16)
        ext_ref[pl.ds(0, 2 * W), :] = zhalo
        for b in range(Nb):
            ext_ref[pl.ds(2 * W + b * S + HW, W), :] = zhalo[:W]
        ext_ref[pl.ds(2 * W + Nb * S, W), :] = zhalo[:W]

        k2 = k2_ref[...]
        for b in range(Nb):
            sk = jnp.transpose(skip_ref[b])
            xt = jnp.transpose(x_ref[pl.ds(b * Cx, Cx), :])
            up = jnp.dot(k2, xt, preferred_element_type=jnp.float32)
            act = jnp.concatenate([sk, up], axis=1)
            xs_ref[pl.ds(b * HW, HW), :] = act
            store_slabs(b, act)

        h = conv3x3(Cin, w1a_ref)
        group_norm(h, g1a_ref, b1a_ref,
                   lambda b, y: store_slabs(b, _gelu(y)))
        h = conv3x3(Cin, w1b_ref)
        group_norm(
            h, g1b_ref, b1b_ref,
            lambda b, y: store_slabs(
                b, _gelu(xs_ref[pl.ds(b * HW, HW), :] + y)))

        m = conv3x3(Cin, w2a_ref)
        group_norm(m, g2a_ref, b2a_ref,
                   lambda b, y: store_slabs(b, _gelu(y)))
        o = conv3x3(Cmid, w2b_ref)

        def final(b, y):
            out_ref[b] = jnp.transpose(
                y + emb_ref[b:b + 1, :]).astype(out_ref.dtype)
        group_norm(o, g2b_ref, b2b_ref, final)

    return body, B, K3


def _bilinear_matrix(Hin, Win, H, W):
    def axis_w(n_in, n_out):
        src = jnp.arange(n_out, dtype=jnp.float32) * (n_in - 1) / (n_out - 1)
        lo = jnp.clip(jnp.floor(src).astype(jnp.int32), 0, n_in - 2)
        frac = src - lo.astype(jnp.float32)
        i = jnp.arange(n_in)[None, :]
        wlo = (i == lo[:, None]) * (1.0 - frac)[:, None]
        whi = (i == (lo + 1)[:, None]) * frac[:, None]
        return wlo + whi

    uh = axis_w(Hin, H)
    uw = axis_w(Win, W)
    k2 = uh[:, None, :, None] * uw[None, :, None, :]
    return k2.reshape(H * W, Hin * Win)


@jax.jit
def kernel(w1a, w1b, w2a, w2b, g1a, b1a, g1b, b1b,
           g2a, b2a, g2b, b2b, wlin, blin, x, skip_x, embeddings):
    N, Cx, Hin, Win = x.shape
    _, Cs, H, W = skip_x.shape
    Cin = Cx + Cs
    HW = H * W
    HWin = Hin * Win

    x2d = x.reshape(N * Cx, HWin).astype(jnp.float32)
    skip3 = skip_x.reshape(N, Cs, HW).astype(jnp.float32)
    k2 = _bilinear_matrix(Hin, Win, H, W)

    Cmid = w2a.shape[-1]
    Cout = w2b.shape[-1]

    wb1a = w1a.reshape(9 * Cin, Cin).astype(jnp.bfloat16)
    wb1b = w1b.reshape(9 * Cin, Cin).astype(jnp.bfloat16)
    wb2a = w2a.reshape(9 * Cin, Cmid).astype(jnp.bfloat16)
    wb2b = w2b.reshape(9 * Cmid, Cout).astype(jnp.bfloat16)

    ga1a = g1a.reshape(1, Cin)
    bb1a = b1a.reshape(1, Cin)
    ga1b = g1b.reshape(1, Cin)
    bb1b = b1b.reshape(1, Cin)
    ga2a = g2a.reshape(1, Cmid)
    bb2a = b2a.reshape(1, Cmid)
    ga2b = g2b.reshape(1, Cout)
    bb2b = b2b.reshape(1, Cout)

    e = embeddings.astype(jnp.float32)
    e = e * jax.nn.sigmoid(e)
    emb = e @ wlin + blin

    Nb = _batch_block(N)
    M = Nb * HW
    body, B, K3 = _make_body(Nb, H, W, Cx, Cs, Cmid, Cout)

    def full(a):
        nd = a.ndim
        return pl.BlockSpec(a.shape, lambda b: (0,) * nd)

    out = pl.pallas_call(
        body,
        out_shape=jax.ShapeDtypeStruct((N, Cout, HW), jnp.float32),
        grid=(N // Nb,),
        in_specs=[
            pl.BlockSpec((Nb * Cx, HWin), lambda b: (b, 0)),
            pl.BlockSpec((Nb, Cs, HW), lambda b: (b, 0, 0)),
            full(k2),
            full(wb1a), full(wb1b), full(wb2a), full(wb2b),
            full(ga1a), full(bb1a), full(ga1b), full(bb1b),
            full(ga2a), full(bb2a), full(ga2b), full(bb2b),
            pl.BlockSpec((Nb, Cout), lambda b: (b, 0)),
        ],
        out_specs=pl.BlockSpec((Nb, Cout, HW), lambda b: (b, 0, 0)),
        scratch_shapes=[pltpu.VMEM((B, K3), jnp.bfloat16),
                        pltpu.VMEM((M, Cin), jnp.float32)],
        compiler_params=pltpu.CompilerParams(
            dimension_semantics=("parallel",)),
    )(x2d, skip3, k2, wb1a, wb1b, wb2a, wb2b,
      ga1a, bb1a, ga1b, bb1b, ga2a, bb2a, ga2b, bb2b, emb)

    return out.reshape(N, Cout, H, W)
